# Initial kernel scaffold; baseline (speedup 1.0000x reference)
import jax
import jax.numpy as jnp
from jax.experimental import pallas as pl


def kernel(x, edge_index, edge_attr, W1, We, Wu, Wn, Weo):
    raise NotImplementedError("write your pallas kernel here")


# speedup vs baseline: 2.6912x; 2.6912x over previous
import functools

import jax
import jax.numpy as jnp
from jax import lax
from jax.experimental import pallas as pl
from jax.experimental.pallas import tpu as pltpu
from jax.experimental.pallas import tpu_sc as plsc

N = 10000
E = 320000
D = 128
DE = 16

NC = 2
NS = 16
NW = NC * NS

CH = 128
CPW = -(-E // (CH * NW))
E_PAD = CPW * NW * CH
NROW = 10240
ZR = NROW // NS

EPW = E // NW
CH2 = 80


def _h_body(x_ref, w_ref, o_ref):
    o_ref[...] = jnp.maximum(
        jnp.dot(x_ref[...], w_ref[...], preferred_element_type=jnp.float32), 0.0)


def _gate_body(ea_ref, we_ref, o_ref):
    o_ref[...] = jax.nn.sigmoid(
        jnp.dot(ea_ref[...], we_ref[...], preferred_element_type=jnp.float32))


def _heads_body(h_ref, part_ref, wu_ref, wn_ref, w4_ref, np_ref, pq_ref):
    agg = part_ref[0, :N, :] + part_ref[1, :N, :]
    h2 = jnp.maximum(
        h_ref[...] + jnp.dot(agg, wu_ref[...], preferred_element_type=jnp.float32),
        0.0)
    np_ref[...] = jnp.dot(h2, wn_ref[...], preferred_element_type=jnp.float32)
    pq_ref[...] = jnp.dot(h2, w4_ref[...], preferred_element_type=jnp.float32)


_MESH = plsc.VectorSubcoreMesh(core_axis_name="c", subcore_axis_name="s")


@functools.partial(
    pl.kernel,
    out_type=jax.ShapeDtypeStruct((NC, NROW, D), jnp.float32),
    mesh=_MESH,
    scratch_types=[
        pltpu.VMEM((CH,), jnp.int32),
        pltpu.VMEM((CH,), jnp.int32),
        pltpu.VMEM((CH, D), jnp.float32),
        pltpu.VMEM((CH, D), jnp.float32),
        pltpu.VMEM_SHARED((NROW, D), jnp.float32),
        pltpu.SemaphoreType.DMA,
    ],
)
def _agg_kernel(h_hbm, gate_hbm, src_hbm, dst_hbm, out_hbm,
                src_v, dst_v, rows_v, gate_v, acc_sh, sem):
    cid = lax.axis_index("c")
    sid = lax.axis_index("s")
    wid = cid * NS + sid

    zero = jnp.zeros((16,), jnp.float32)

    def _zrow(r, _):
        for c in range(D // 16):
            rows_v[r, pl.ds(c * 16, 16)] = zero
        return 0

    lax.fori_loop(0, CH, _zrow, 0)
    zbase = sid * ZR
    for t in range(ZR // CH):
        pltpu.sync_copy(rows_v, acc_sh.at[pl.ds(zbase + t * CH, CH)])
    plsc.subcore_barrier()

    def _chunk(i, _):
        base = (wid * CPW + i) * CH
        pltpu.sync_copy(src_hbm.at[pl.ds(base, CH)], src_v)
        pltpu.sync_copy(dst_hbm.at[pl.ds(base, CH)], dst_v)
        pltpu.async_copy(h_hbm.at[src_v], rows_v, sem).wait()
        pltpu.sync_copy(gate_hbm.at[pl.ds(base, CH)], gate_v)

        def _mul(r, _):
            for c in range(D // 16):
                s = pl.ds(c * 16, 16)
                rows_v[r, s] = rows_v[r, s] * gate_v[r, s]
            return 0

        lax.fori_loop(0, CH, _mul, 0)
        pltpu.sync_copy(rows_v, acc_sh.at[dst_v], add=True)
        return 0

    lax.fori_loop(0, CPW, _chunk, 0)
    plsc.subcore_barrier()

    ob = sid * ZR
    pltpu.sync_copy(acc_sh.at[pl.ds(ob, ZR)], out_hbm.at[cid, pl.ds(ob, ZR)])


@functools.partial(
    pl.kernel,
    out_type=jax.ShapeDtypeStruct((E * 2,), jnp.float32),
    mesh=_MESH,
    scratch_types=[
        pltpu.VMEM((N * 4,), jnp.float32),
        pltpu.VMEM((CH2,), jnp.int32),
        pltpu.VMEM((CH2,), jnp.int32),
        pltpu.VMEM((EPW * 2,), jnp.float32),
    ],
    compiler_params=pltpu.CompilerParams(needs_layout_passes=False),
)
def _edge_kernel(pq_hbm, src_hbm, dst_hbm, out_hbm, pq_v, src_v, dst_v, out_v):
    cid = lax.axis_index("c")
    sid = lax.axis_index("s")
    wid = cid * NS + sid
    ebase = wid * EPW
    pltpu.sync_copy(pq_hbm, pq_v)
    lane = lax.iota(jnp.int32, 16)

    def _chunk(i, _):
        cb = i * CH2
        pltpu.sync_copy(src_hbm.at[pl.ds(ebase + cb, CH2)], src_v)
        pltpu.sync_copy(dst_hbm.at[pl.ds(ebase + cb, CH2)], dst_v)
        for g in range(CH2 // 16):
            si = src_v[pl.ds(g * 16, 16)] * 4
            di = dst_v[pl.ds(g * 16, 16)] * 4
            p0 = plsc.load_gather(pq_v, [si])
            p1 = plsc.load_gather(pq_v, [si + 1])
            q0 = plsc.load_gather(pq_v, [di + 2])
            q1 = plsc.load_gather(pq_v, [di + 3])
            oi = (cb + g * 16 + lane) * 2
            plsc.store_scatter(out_v, [oi], p0 + q0)
            plsc.store_scatter(out_v, [oi + 1], p1 + q1)
        return 0

    lax.fori_loop(0, EPW // CH2, _chunk, 0)
    pltpu.sync_copy(out_v, out_hbm.at[pl.ds(ebase * 2, EPW * 2)])


def kernel(x, edge_index, edge_attr, W1, We, Wu, Wn, Weo):
    src = edge_index[0]
    dst = edge_index[1]
    pad = E_PAD - E
    srcp = jnp.concatenate([src, jnp.zeros((pad,), jnp.int32)])
    dstp = jnp.concatenate([dst, jnp.full((pad,), N, jnp.int32)])
    ea_pad = jnp.concatenate([edge_attr, jnp.zeros((pad, DE), jnp.float32)])

    h = pl.pallas_call(
        _h_body,
        out_shape=jax.ShapeDtypeStruct((N, D), jnp.float32),
    )(x, W1)

    BE = 4096
    gate = pl.pallas_call(
        _gate_body,
        grid=(E_PAD // BE,),
        in_specs=[
            pl.BlockSpec((BE, DE), lambda i: (i, 0)),
            pl.BlockSpec((DE, D), lambda i: (0, 0)),
        ],
        out_specs=pl.BlockSpec((BE, D), lambda i: (i, 0)),
        out_shape=jax.ShapeDtypeStruct((E_PAD, D), jnp.float32),
    )(ea_pad, We)

    partials = _agg_kernel(h, gate, srcp, dstp)

    W4 = jnp.concatenate([Weo[:D], Weo[D:]], axis=1)
    node_pred, pq = pl.pallas_call(
        _heads_body,
        out_shape=(
            jax.ShapeDtypeStruct((N, 2), jnp.float32),
            jax.ShapeDtypeStruct((N, 4), jnp.float32),
        ),
    )(h, partials, Wu, Wn, W4)

    edge_pred = _edge_kernel(pq.reshape(-1), srcp, dstp)
    return node_pred, edge_pred.reshape(E, 2)


# speedup vs baseline: 3.2878x; 1.2217x over previous
import functools

import jax
import jax.numpy as jnp
from jax import lax
from jax.experimental import pallas as pl
from jax.experimental.pallas import tpu as pltpu
from jax.experimental.pallas import tpu_sc as plsc

N = 10000
E = 320000
D = 128
DE = 16

NC = 2
NS = 16
NW = NC * NS

CH = 96
CPW = 106
E_PAD = CPW * NW * CH
NROW = 10112
ZR = NROW // NS

EPW = E // NW


def _h_body(x_ref, w_ref, o_ref):
    o_ref[...] = jnp.maximum(
        jnp.dot(x_ref[...], w_ref[...], preferred_element_type=jnp.float32), 0.0)


def _gate_body(ea_ref, we_ref, o_ref):
    o_ref[...] = jax.nn.sigmoid(
        jnp.dot(ea_ref[...], we_ref[...], preferred_element_type=jnp.float32))


def _heads_body(h_ref, part_ref, wu_ref, wn_ref, w4_ref, np_ref, pq_ref):
    agg = part_ref[0, :N, :] + part_ref[1, :N, :]
    h2 = jnp.maximum(
        h_ref[...] + jnp.dot(agg, wu_ref[...], preferred_element_type=jnp.float32),
        0.0)
    np_ref[...] = jnp.dot(h2, wn_ref[...], preferred_element_type=jnp.float32)
    pq_ref[...] = jnp.dot(h2, w4_ref[...], preferred_element_type=jnp.float32)


_MESH = plsc.VectorSubcoreMesh(core_axis_name="c", subcore_axis_name="s")


@functools.partial(
    pl.kernel,
    out_type=jax.ShapeDtypeStruct((NC, NROW, D), jnp.float32),
    mesh=_MESH,
    scratch_types=[
        pltpu.VMEM((CH, D), jnp.float32),
        pltpu.VMEM((CH, D), jnp.float32),
        pltpu.VMEM((CH, D), jnp.float32),
        pltpu.VMEM((CH, D), jnp.float32),
        pltpu.VMEM((CH,), jnp.int32),
        pltpu.VMEM((CH,), jnp.int32),
        pltpu.VMEM((CH,), jnp.int32),
        pltpu.VMEM((CH,), jnp.int32),
        pltpu.VMEM((CH,), jnp.int32),
        pltpu.VMEM((CH,), jnp.int32),
        pltpu.VMEM_SHARED((NROW, D), jnp.float32),
        pltpu.SemaphoreType.DMA,
        pltpu.SemaphoreType.DMA,
        pltpu.SemaphoreType.DMA,
        pltpu.SemaphoreType.DMA,
        pltpu.SemaphoreType.DMA,
        pltpu.SemaphoreType.DMA,
        pltpu.SemaphoreType.DMA,
        pltpu.SemaphoreType.DMA,
        pltpu.SemaphoreType.DMA,
    ],
)
def _agg_kernel(h_hbm, gate_hbm, src_hbm, dst_hbm, out_hbm,
                rows0_v, rows1_v, gate0_v, gate1_v,
                gi0_v, gi1_v, di0_v, di1_v, ss0_v, ss1_v, acc_sh,
                sem_r0, sem_r1, sem_g0, sem_g1, sem_s0, sem_s1,
                sem_i0, sem_i1, sem_m):
    cid = lax.axis_index("c")
    sid = lax.axis_index("s")
    wid = cid * NS + sid
    rows_v = (rows0_v, rows1_v)
    gate_v = (gate0_v, gate1_v)
    gi_v = (gi0_v, gi1_v)
    di_v = (di0_v, di1_v)
    ss_v = (ss0_v, ss1_v)
    sem_r = (sem_r0, sem_r1)
    sem_g = (sem_g0, sem_g1)
    sem_s = (sem_s0, sem_s1)
    sem_i = (sem_i0, sem_i1)

    ibase = wid * CPW

    def _fire_idx(i, b):
        pltpu.async_copy(src_hbm.at[pl.ds((ibase + i) * CH, CH)], gi_v[b],
                         sem_i[b])
        pltpu.async_copy(dst_hbm.at[pl.ds((ibase + i) * CH, CH)], di_v[b],
                         sem_i[b])

    def _wait_idx(i, b):
        pltpu.make_async_copy(src_hbm.at[pl.ds((ibase + i) * CH, CH)], gi_v[b],
                              sem_i[b]).wait()
        pltpu.make_async_copy(dst_hbm.at[pl.ds((ibase + i) * CH, CH)], di_v[b],
                              sem_i[b]).wait()

    def _fire(i, b):
        pltpu.async_copy(h_hbm.at[gi_v[b]], rows_v[b], sem_r[b])
        pltpu.async_copy(gate_hbm.at[pl.ds((ibase + i) * CH, CH)],
                         gate_v[b], sem_g[b])

    def _wait_data(i, b):
        pltpu.make_async_copy(h_hbm.at[gi_v[b]], rows_v[b], sem_r[b]).wait()
        pltpu.make_async_copy(gate_hbm.at[pl.ds((ibase + i) * CH, CH)],
                              gate_v[b], sem_g[b]).wait()

    def _wait_scatter(i, b):
        pltpu.make_async_copy(rows_v[b], acc_sh.at[ss_v[b]], sem_s[b]).wait()

    _fire_idx(0, 0)
    _fire_idx(1, 1)

    zero = jnp.zeros((16,), jnp.float32)

    def _zrow(r, _):
        for c in range(D // 16):
            rows0_v[r, pl.ds(c * 16, 16)] = zero
        return 0

    lax.fori_loop(0, CH, _zrow, 0)
    zbase = sid * ZR
    zcp = [
        pltpu.make_async_copy(rows0_v, acc_sh.at[pl.ds(zbase + t * CH, CH)],
                              sem_m)
        for t in range(ZR // CH)
    ] + [
        pltpu.make_async_copy(rows0_v.at[pl.ds(0, ZR % CH)],
                              acc_sh.at[pl.ds(zbase + (ZR // CH) * CH, ZR % CH)],
                              sem_m)
    ]
    for c in zcp:
        c.start()
    for c in zcp:
        c.wait()
    plsc.subcore_barrier()

    _wait_idx(0, 0)
    _fire(0, 0)

    def _pair(j, _):
        for b in range(2):
            i = j * 2 + b
            nb = 1 - b
            _wait_data(i, b)

            @pl.when(i + 1 < CPW)
            def _():
                @pl.when(i >= 1)
                def _():
                    _wait_scatter(i - 1, nb)
                _wait_idx(i + 1, nb)
                _fire(i + 1, nb)

            for c in range(CH // 16):
                s = pl.ds(c * 16, 16)
                ss_v[b][s] = di_v[b][s]

            @pl.when(i + 2 < CPW)
            def _():
                _fire_idx(i + 2, b)

            def _mul(r, _):
                for c in range(D // 16):
                    s = pl.ds(c * 16, 16)
                    rows_v[b][r, s] = rows_v[b][r, s] * gate_v[b][r, s]
                return 0

            lax.fori_loop(0, CH, _mul, 0)
            pltpu.async_copy(rows_v[b], acc_sh.at[ss_v[b]], sem_s[b], add=True)
        return 0

    lax.fori_loop(0, CPW // 2, _pair, 0)
    _wait_scatter(CPW - 2, 0)
    _wait_scatter(CPW - 1, 1)
    plsc.subcore_barrier()

    ob = sid * ZR
    pltpu.sync_copy(acc_sh.at[pl.ds(ob, ZR)], out_hbm.at[cid, pl.ds(ob, ZR)])


@functools.partial(
    pl.kernel,
    out_type=jax.ShapeDtypeStruct((E * 2,), jnp.float32),
    mesh=_MESH,
    scratch_types=[
        pltpu.VMEM((N * 4,), jnp.float32),
        pltpu.VMEM((EPW,), jnp.int32),
        pltpu.VMEM((EPW,), jnp.int32),
        pltpu.VMEM((EPW * 2,), jnp.float32),
        pltpu.SemaphoreType.DMA,
    ],
    compiler_params=pltpu.CompilerParams(needs_layout_passes=False),
)
def _edge_kernel(pq_hbm, src_hbm, dst_hbm, out_hbm,
                 pq_v, src_v, dst_v, out_v, sem):
    cid = lax.axis_index("c")
    sid = lax.axis_index("s")
    wid = cid * NS + sid
    ebase = wid * EPW
    pltpu.async_copy(pq_hbm, pq_v, sem)
    pltpu.async_copy(src_hbm.at[pl.ds(ebase, EPW)], src_v, sem)
    pltpu.async_copy(dst_hbm.at[pl.ds(ebase, EPW)], dst_v, sem)
    pltpu.make_async_copy(pq_hbm, pq_v, sem).wait()
    pltpu.make_async_copy(src_hbm.at[pl.ds(ebase, EPW)], src_v, sem).wait()
    pltpu.make_async_copy(dst_hbm.at[pl.ds(ebase, EPW)], dst_v, sem).wait()
    lane = lax.iota(jnp.int32, 16)

    def _grp(g, _):
        gb = g * 16
        si = src_v[pl.ds(gb, 16)] * 4
        di = dst_v[pl.ds(gb, 16)] * 4
        p0 = plsc.load_gather(pq_v, [si])
        p1 = plsc.load_gather(pq_v, [si + 1])
        q0 = plsc.load_gather(pq_v, [di + 2])
        q1 = plsc.load_gather(pq_v, [di + 3])
        oi = (gb + lane) * 2
        plsc.store_scatter(out_v, [oi], p0 + q0)
        plsc.store_scatter(out_v, [oi + 1], p1 + q1)
        return 0

    lax.fori_loop(0, EPW // 16, _grp, 0)
    pltpu.sync_copy(out_v, out_hbm.at[pl.ds(ebase * 2, EPW * 2)])


def kernel(x, edge_index, edge_attr, W1, We, Wu, Wn, Weo):
    src = edge_index[0]
    dst = edge_index[1]
    pad = E_PAD - E
    srcp = jnp.concatenate([src, jnp.zeros((pad,), jnp.int32)])
    dstp = jnp.concatenate([dst, jnp.full((pad,), N, jnp.int32)])
    ea_pad = jnp.concatenate([edge_attr, jnp.zeros((pad, DE), jnp.float32)])

    h = pl.pallas_call(
        _h_body,
        out_shape=jax.ShapeDtypeStruct((N, D), jnp.float32),
    )(x, W1)

    BE = 3072
    gate = pl.pallas_call(
        _gate_body,
        grid=(E_PAD // BE,),
        in_specs=[
            pl.BlockSpec((BE, DE), lambda i: (i, 0)),
            pl.BlockSpec((DE, D), lambda i: (0, 0)),
        ],
        out_specs=pl.BlockSpec((BE, D), lambda i: (i, 0)),
        out_shape=jax.ShapeDtypeStruct((E_PAD, D), jnp.float32),
    )(ea_pad, We)

    partials = _agg_kernel(h, gate, srcp, dstp)

    W4 = jnp.concatenate([Weo[:D], Weo[D:]], axis=1)
    node_pred, pq = pl.pallas_call(
        _heads_body,
        out_shape=(
            jax.ShapeDtypeStruct((N, 2), jnp.float32),
            jax.ShapeDtypeStruct((N, 4), jnp.float32),
        ),
    )(h, partials, Wu, Wn, W4)

    edge_pred = _edge_kernel(pq.reshape(-1), src, dst)
    return node_pred, edge_pred.reshape(E, 2)
